# baseline (speedup 1.0000x reference)
import functools

import jax
import jax.numpy as jnp
from jax import lax
from jax.experimental import pallas as pl
from jax.experimental.pallas import tpu as pltpu

_MU = 0.5


def _loss_kernel(out_ref, pert_ref, y_ref, acc_ref, *, mu, inv_n):
    i = pl.program_id(0)
    logits = out_ref[...] + mu * pert_ref[...]
    m = jnp.max(logits, axis=-1, keepdims=True)
    t = logits - m
    s = jnp.sum(jnp.exp(t), axis=-1, keepdims=True)
    c = logits.shape[1]
    col = lax.broadcasted_iota(jnp.int32, (1, c), 1)
    y_col = jnp.transpose(y_ref[...], (1, 0))
    tgt_m = jnp.sum(jnp.where(col == y_col, t, 0.0), axis=-1, keepdims=True)
    part = jnp.sum(jnp.log(s) - tgt_m, keepdims=True) * inv_n

    @pl.when(i == 0)
    def _():
        acc_ref[...] = jnp.zeros_like(acc_ref)

    acc_ref[...] += part


def kernel(output, perturbation, y):
    n, c = output.shape
    tile_n = n
    for cand in (512, 256, 128, 64, 32, 16, 8):
        if n % cand == 0:
            tile_n = cand
            break
    y_row = y.astype(jnp.int32).reshape(1, n)
    loss = pl.pallas_call(
        functools.partial(_loss_kernel, mu=_MU, inv_n=1.0 / n),
        grid=(n // tile_n,),
        in_specs=[
            pl.BlockSpec((tile_n, c), lambda i: (i, 0)),
            pl.BlockSpec((tile_n, c), lambda i: (i, 0)),
            pl.BlockSpec((1, tile_n), lambda i: (0, i)),
        ],
        out_specs=pl.BlockSpec((1, 1), lambda i: (0, 0)),
        out_shape=jax.ShapeDtypeStruct((1, 1), jnp.float32),
        compiler_params=pltpu.CompilerParams(
            dimension_semantics=("arbitrary",),
            vmem_limit_bytes=48 * 1024 * 1024),
    )(output, perturbation, y_row)
    return loss[0, 0]


# speedup vs baseline: 1.0822x; 1.0822x over previous
import functools

import jax
import jax.numpy as jnp
from jax import lax
from jax.experimental import pallas as pl
from jax.experimental.pallas import tpu as pltpu

_MU = 0.5


def _loss_kernel(out_ref, pert_ref, y_ref, acc_ref, *, mu, inv_n):
    i = pl.program_id(0)
    logits = out_ref[...] + mu * pert_ref[...]
    m = jnp.max(logits, axis=-1, keepdims=True)
    t = logits - m
    s = jnp.sum(jnp.exp(t), axis=-1, keepdims=True)
    c = logits.shape[1]
    col = lax.broadcasted_iota(jnp.int32, (1, c), 1)
    y_col = jnp.transpose(y_ref[...], (1, 0))
    tgt_m = jnp.sum(jnp.where(col == y_col, t, 0.0), axis=-1, keepdims=True)
    part = jnp.sum(jnp.log(s) - tgt_m, keepdims=True) * inv_n

    @pl.when(i == 0)
    def _():
        acc_ref[...] = jnp.zeros_like(acc_ref)

    acc_ref[...] += part


def kernel(output, perturbation, y):
    n, c = output.shape
    tile_n = n
    for cand in (1024, 512, 256, 128, 64, 32, 16, 8):
        if n % cand == 0:
            tile_n = cand
            break
    y_row = y.astype(jnp.int32).reshape(1, n)
    loss = pl.pallas_call(
        functools.partial(_loss_kernel, mu=_MU, inv_n=1.0 / n),
        grid=(n // tile_n,),
        in_specs=[
            pl.BlockSpec((tile_n, c), lambda i: (i, 0)),
            pl.BlockSpec((tile_n, c), lambda i: (i, 0)),
            pl.BlockSpec((1, tile_n), lambda i: (0, i)),
        ],
        out_specs=pl.BlockSpec((1, 1), lambda i: (0, 0)),
        out_shape=jax.ShapeDtypeStruct((1, 1), jnp.float32),
        compiler_params=pltpu.CompilerParams(
            dimension_semantics=("arbitrary",),
            vmem_limit_bytes=48 * 1024 * 1024),
    )(output, perturbation, y_row)
    return loss[0, 0]
